# Initial kernel scaffold; baseline (speedup 1.0000x reference)
import jax
import jax.numpy as jnp
from jax.experimental import pallas as pl


def kernel(x, edge_index, edge_attr, params):
    raise NotImplementedError("write your pallas kernel here")


# speedup vs baseline: 1.8233x; 1.8233x over previous
import functools

import jax
import jax.numpy as jnp
from jax.experimental import pallas as pl
from jax.experimental.pallas import tpu as pltpu

N = 100000
E = 1600000
HID = 32


def _sub_body(x_ref, x0_ref, e_ref, o_ref, eo_ref):
    o_ref[...] = x_ref[...] - x0_ref[...]
    eo_ref[...] = e_ref[...]


def _finalize(xx, emb):
    blk = 2000
    out, emb_o = pl.pallas_call(
        _sub_body,
        grid=(N // blk,),
        in_specs=[
            pl.BlockSpec((blk, 1), lambda i: (i, 0)),
            pl.BlockSpec((1, 1), lambda i: (0, 0)),
            pl.BlockSpec((blk, HID), lambda i: (i, 0)),
        ],
        out_specs=[
            pl.BlockSpec((blk, 1), lambda i: (i, 0)),
            pl.BlockSpec((blk, HID), lambda i: (i, 0)),
        ],
        out_shape=[
            jax.ShapeDtypeStruct((N, 1), jnp.float32),
            jax.ShapeDtypeStruct((N, HID), jnp.float32),
        ],
    )(xx, xx[0:1], emb)
    return out, emb_o


def _gat(x, src, dst, ea, p, self_loop, loop_attr):
    h = x @ p['W']
    asrc = (h * p['att_src']).sum(-1)
    adst = (h * p['att_dst']).sum(-1)
    he = ea @ p['We']
    ae = (he * p['att_edge']).sum(-1)
    alpha = asrc[src] + adst[dst] + ae
    alpha = jnp.where(alpha >= 0, alpha, 0.2 * alpha)
    a = jnp.exp(alpha)
    s = jax.ops.segment_sum(a, dst, num_segments=N)
    acc = jax.ops.segment_sum(h[src] * a[:, None], dst, num_segments=N)
    if self_loop:
        he_l = loop_attr @ p['We']
        ael = (he_l * p['att_edge']).sum(-1)
        alpha_s = asrc + adst + ael
        alpha_s = jnp.where(alpha_s >= 0, alpha_s, 0.2 * alpha_s)
        es = jnp.exp(alpha_s)
        acc = acc + es[:, None] * h
        s = s + es
    out = acc / (s + 1e-16)[:, None]
    return out + p['b']


def kernel(x, edge_index, edge_attr, params):
    src_u, dst_u = edge_index[0], edge_index[1]
    src_d, dst_d = edge_index[1], edge_index[0]
    cnt = jax.ops.segment_sum(jnp.ones(E, jnp.float32), dst_u, num_segments=N)
    ea_sum = jax.ops.segment_sum(edge_attr, dst_u, num_segments=N)
    loop_attr = ea_sum / jnp.maximum(cnt, 1.0)[:, None]
    xu = x
    for i in range(3):
        xu = jax.nn.relu(_gat(xu, src_u, dst_u, edge_attr,
                              params['toup'][i], True, loop_attr))
    xd = jax.nn.relu(_gat(xu, src_d, dst_d, edge_attr,
                          params['todown'][0], False, None))
    xx = xd + xu
    for i in range(2):
        xx = jax.nn.relu(xx @ params['lin'][i]['W'] + params['lin'][i]['b'])
        xd = jax.nn.relu(_gat(xx, src_d, dst_d, edge_attr,
                              params['todown'][i], False, None))
        xx = xd + xu
    final_emb = xx
    out_lin = xx @ params['lin'][-1]['W'] + params['lin'][-1]['b']
    out, emb = _finalize(out_lin, final_emb)
    return (out.T, emb[None, :, :])


# speedup vs baseline: 32.2062x; 17.6635x over previous
import functools

import jax
import jax.numpy as jnp
from jax import lax
from jax.experimental import pallas as pl
from jax.experimental.pallas import tpu as pltpu
from jax.experimental.pallas import tpu_sc as plsc

N = 100000
NP = 100352
NT = NP // 16
E = 1600000
EROWS = E // 128
ERP = 12544
RPT = ERP // 16
CH = 2
NCHUNK = RPT // CH
HID = 32
HALF = 16
NEG = -1.0e30

_f32 = jnp.float32
_i32 = jnp.int32

_MESH = plsc.VectorSubcoreMesh(core_axis_name="c", subcore_axis_name="s",
                               num_cores=2, num_subcores=16)

_GDN = lax.GatherDimensionNumbers(offset_dims=(), collapsed_slice_dims=(0,),
                                  start_index_map=(0,))


def _bcast_lane(v16, l):
    idx = jnp.full((16, 1), l, _i32)
    return lax.gather(v16, idx, _GDN, (1,),
                      mode=lax.GatherScatterMode.PROMISE_IN_BOUNDS)


def _zero_fill(z2, z1):
    def zz(i, carry):
        z2[i, :] = jnp.zeros((HALF,), _f32)
        return carry
    lax.fori_loop(0, 392, zz, 0)
    for i in range(25):
        z1[pl.ds(i * 16, 16)] = jnp.zeros((16,), _f32)


def _deg_body(dst2, ones2, ea3, easum_o, cnt_o,
              easum_sh, cnt_sh, dst_v, ones_v, rows_v, z2, z1, sem_r):
    c = lax.axis_index("c")
    s = lax.axis_index("s")
    nbase = s * NT
    _zero_fill(z2, z1)

    def zo(k, carry):
        pltpu.sync_copy(z2, easum_sh.at[pl.ds(nbase + k * 392, 392)])
        pltpu.sync_copy(z1.at[pl.ds(0, 392)],
                        cnt_sh.at[pl.ds(nbase + k * 392, 392)])
        return carry
    lax.fori_loop(0, 16, zo, 0)
    plsc.subcore_barrier()

    base_row = (c * 16 + s) * (ERP // 32)

    def chunk(ci, carry):
        row0 = base_row + ci * CH
        pltpu.sync_copy(dst2.at[pl.ds(row0, CH)], dst_v)
        pltpu.sync_copy(ones2.at[pl.ds(row0, CH)], ones_v)
        pltpu.async_copy(ea3.at[pl.ds(row0 * 128, CH * 128)], rows_v,
                         sem_r).wait()
        def rsc(j, carry2):
            pltpu.sync_copy(rows_v.at[pl.ds(j * 128, 128)],
                            easum_sh.at[dst_v.at[j]], add=True)
            pltpu.sync_copy(ones_v.at[j], cnt_sh.at[dst_v.at[j]], add=True)
            return carry2
        lax.fori_loop(0, CH, rsc, 0)
        return carry
    lax.fori_loop(0, (ERP // 32) // CH, chunk, 0)

    plsc.subcore_barrier()
    obase = c * NP + nbase
    pltpu.sync_copy(easum_sh.at[pl.ds(nbase, NT)], easum_o.at[pl.ds(obase, NT)])
    pltpu.sync_copy(cnt_sh.at[pl.ds(nbase, NT)], cnt_o.at[pl.ds(obase, NT)])


_deg_kernel = functools.partial(
    pl.kernel,
    compiler_params=pltpu.CompilerParams(use_tc_tiling_on_sc=False, needs_layout_passes=False),
    out_type=[jax.ShapeDtypeStruct((2 * NP, HALF), _f32),
              jax.ShapeDtypeStruct((2 * NP,), _f32)],
    mesh=_MESH,
    scratch_types=[
        pltpu.VMEM_SHARED((NP, HALF), _f32),
        pltpu.VMEM_SHARED((NP,), _f32),
        pltpu.VMEM((CH, 128), _i32),
        pltpu.VMEM((CH, 128), _f32),
        pltpu.VMEM((CH * 128, HALF), _f32),
        pltpu.VMEM((392, HALF), _f32),
        pltpu.VMEM((400,), _f32),
        pltpu.SemaphoreType.DMA,
    ],
)(_deg_body)


def _edge_body(src2, dst2, ae2, asrc, adst, hflat,
               outh, s_out,
               out_sh, s_sh, asrc_sh, adst_sh,
               src_v, dst_v, ae_v, si_v, as_v, ad_v, a_v, rows_v, z2, z1,
               sem_g, sem_r):
    c = lax.axis_index("c")
    s = lax.axis_index("s")
    nbase = s * NT

    def zz(i, carry):
        z2[i, :] = jnp.zeros((HALF,), _f32)
        return carry
    lax.fori_loop(0, 56, zz, 0)

    def z1f(i, carry):
        z1[pl.ds(i * 16, 16)] = jnp.zeros((16,), _f32)
        return carry
    lax.fori_loop(0, 25, z1f, 0)

    def zo(k, carry):
        pltpu.sync_copy(z2, out_sh.at[pl.ds(nbase + k * 56, 56)])
        return carry
    lax.fori_loop(0, 112, zo, 0)

    def zs(k, carry):
        pltpu.sync_copy(z1.at[pl.ds(0, 392)],
                        s_sh.at[pl.ds(nbase + k * 392, 392)])
        return carry
    lax.fori_loop(0, 16, zs, 0)
    pltpu.sync_copy(asrc.at[pl.ds(nbase, NT)], asrc_sh.at[pl.ds(nbase, NT)])
    pltpu.sync_copy(adst.at[pl.ds(nbase, NT)], adst_sh.at[pl.ds(nbase, NT)])
    plsc.subcore_barrier()

    coff = c * N

    def chunk(ci, carry):
        row0 = s * RPT + ci * CH
        pltpu.sync_copy(src2.at[pl.ds(row0, CH)], src_v)
        pltpu.sync_copy(dst2.at[pl.ds(row0, CH)], dst_v)
        pltpu.sync_copy(ae2.at[pl.ds(row0, CH)], ae_v)

        def off(jg, carry2):
            j = jg // 8
            g = (jg % 8) * 16
            si_v[j, pl.ds(g, 16)] = src_v[j, pl.ds(g, 16)] + coff
            return carry2
        lax.fori_loop(0, CH * 8, off, 0)

        dg = []
        for j in range(CH):
            dg.append(pltpu.async_copy(asrc_sh.at[src_v.at[j]],
                                       as_v.at[j], sem_g))
            dg.append(pltpu.async_copy(adst_sh.at[dst_v.at[j]],
                                       ad_v.at[j], sem_g))
        dr = [pltpu.async_copy(hflat.at[si_v.at[j]], rows_v.at[j], sem_r)
              for j in range(CH)]
        for d in dg:
            d.wait()

        def comp(jg, carry2):
            j = jg // 8
            g = (jg % 8) * 16
            v = (as_v[j, pl.ds(g, 16)] + ad_v[j, pl.ds(g, 16)]
                 + ae_v[j, pl.ds(g, 16)])
            v = jnp.where(v >= 0, v, 0.2 * v)
            a_v[j, pl.ds(g, 16)] = jnp.exp(v)
            return carry2
        lax.fori_loop(0, CH * 8, comp, 0)

        @pl.when(c == 0)
        def _():
            def ssc(j, carry2):
                pltpu.sync_copy(a_v.at[j], s_sh.at[dst_v.at[j]], add=True)
                return carry2
            lax.fori_loop(0, CH, ssc, 0)

        for d in dr:
            d.wait()

        def wgt(jg, carry2):
            j = jg // 8
            g = (jg % 8) * 16
            w16 = a_v[j, pl.ds(g, 16)]
            for l in range(16):
                wb = _bcast_lane(w16, l)
                r = rows_v[j, g + l, :]
                rows_v[j, g + l, :] = r * wb
            return carry2
        lax.fori_loop(0, CH * 8, wgt, 0)

        def rsc(j, carry2):
            pltpu.sync_copy(rows_v.at[j], out_sh.at[dst_v.at[j]], add=True)
            return carry2
        lax.fori_loop(0, CH, rsc, 0)
        return carry
    lax.fori_loop(0, NCHUNK, chunk, 0)

    plsc.subcore_barrier()
    obase = c * NP + nbase
    pltpu.sync_copy(out_sh.at[pl.ds(nbase, NT)], outh.at[pl.ds(obase, NT)])

    @pl.when(c == 0)
    def _():
        pltpu.sync_copy(s_sh.at[pl.ds(nbase, NT)], s_out.at[pl.ds(nbase, NT)])


_edge_kernel = functools.partial(
    pl.kernel,
    compiler_params=pltpu.CompilerParams(use_tc_tiling_on_sc=False,
                                         needs_layout_passes=False),
    out_type=[jax.ShapeDtypeStruct((2 * NP, HALF), _f32),
              jax.ShapeDtypeStruct((NP,), _f32)],
    mesh=_MESH,
    scratch_types=[
        pltpu.VMEM_SHARED((NP, HALF), _f32),
        pltpu.VMEM_SHARED((NP,), _f32),
        pltpu.VMEM_SHARED((NP,), _f32),
        pltpu.VMEM_SHARED((NP,), _f32),
        pltpu.VMEM((CH, 128), _i32),
        pltpu.VMEM((CH, 128), _i32),
        pltpu.VMEM((CH, 128), _f32),
        pltpu.VMEM((CH, 128), _i32),
        pltpu.VMEM((CH, 128), _f32),
        pltpu.VMEM((CH, 128), _f32),
        pltpu.VMEM((CH, 128), _f32),
        pltpu.VMEM((CH, 128, HALF), _f32),
        pltpu.VMEM((56, HALF), _f32),
        pltpu.VMEM((400,), _f32),
        pltpu.SemaphoreType.DMA,
        pltpu.SemaphoreType.DMA,
    ],
)(_edge_body)


def _sub_body(x_ref, x0_ref, e_ref, o_ref, eo_ref):
    o_ref[...] = x_ref[...] - x0_ref[...]
    eo_ref[...] = e_ref[...]


def _finalize(xx, emb):
    blk = 2000
    out, emb_o = pl.pallas_call(
        _sub_body,
        grid=(N // blk,),
        in_specs=[
            pl.BlockSpec((blk, 1), lambda i: (i, 0)),
            pl.BlockSpec((1, 1), lambda i: (0, 0)),
            pl.BlockSpec((blk, HID), lambda i: (i, 0)),
        ],
        out_specs=[
            pl.BlockSpec((blk, 1), lambda i: (i, 0)),
            pl.BlockSpec((blk, HID), lambda i: (i, 0)),
        ],
        out_shape=[
            jax.ShapeDtypeStruct((N, 1), jnp.float32),
            jax.ShapeDtypeStruct((N, HID), jnp.float32),
        ],
    )(xx, xx[0:1], emb)
    return out, emb_o


def _pad_rows(a2, val):
    pad = jnp.full((ERP - EROWS, 128), val, a2.dtype)
    return jnp.concatenate([a2, pad], axis=0)


def _padn(v):
    return jnp.concatenate([v, jnp.zeros((NP - N,), v.dtype)])


def _gat_sc(x, su2, du2, ae2, p, self_loop, loop_attr):
    h = x @ p['W']
    asrc = (h * p['att_src']).sum(-1)
    adst = (h * p['att_dst']).sum(-1)
    hflat = jnp.concatenate([h[:, :HALF], h[:, HALF:]], axis=0)
    outh, s_acc = _edge_kernel(su2, du2, ae2, _padn(asrc), _padn(adst), hflat)
    acc = jnp.concatenate([outh[:NP][:N], outh[NP:][:N]], axis=1)
    s_acc = s_acc[:N]
    if self_loop:
        he_l = loop_attr @ p['We']
        ael = (he_l * p['att_edge']).sum(-1)
        alpha_s = asrc + adst + ael
        alpha_s = jnp.where(alpha_s >= 0, alpha_s, 0.2 * alpha_s)
        es = jnp.exp(alpha_s)
        acc = acc + es[:, None] * h
        s_acc = s_acc + es
    out = acc / (s_acc + 1e-16)[:, None]
    return out + p['b']


def _aedge2(ea, p):
    he = ea @ p['We']
    ae = (he * p['att_edge']).sum(-1)
    return _pad_rows(ae.reshape(EROWS, 128), NEG)


def kernel(x, edge_index, edge_attr, params):
    src_u, dst_u = edge_index[0], edge_index[1]
    su2 = _pad_rows(src_u.reshape(EROWS, 128), 0)
    du2 = _pad_rows(dst_u.reshape(EROWS, 128), 0)
    ones2 = _pad_rows(jnp.ones((EROWS, 128), _f32), 0.0)
    ea3 = jnp.concatenate(
        [edge_attr, jnp.zeros((ERP * 128 - E, HALF), _f32)], axis=0)

    easum_p, cnt_p = _deg_kernel(du2, ones2, ea3)
    ea_sum = (easum_p[:NP] + easum_p[NP:])[:N]
    cnt = (cnt_p[:NP] + cnt_p[NP:])[:N]
    loop_attr = ea_sum / jnp.maximum(cnt, 1.0)[:, None]

    xu = x
    for i in range(3):
        p = params['toup'][i]
        xu = jax.nn.relu(_gat_sc(xu, su2, du2, _aedge2(edge_attr, p),
                                 p, True, loop_attr))
    p0 = params['todown'][0]
    ae_d0 = _aedge2(edge_attr, p0)
    xd = jax.nn.relu(_gat_sc(xu, du2, su2, ae_d0, p0, False, None))
    xx = xd + xu
    for i in range(2):
        xx = jax.nn.relu(xx @ params['lin'][i]['W'] + params['lin'][i]['b'])
        p = params['todown'][i]
        ae_d = ae_d0 if i == 0 else _aedge2(edge_attr, p)
        xd = jax.nn.relu(_gat_sc(xx, du2, su2, ae_d, p, False, None))
        xx = xd + xu
    final_emb = xx
    out_lin = xx @ params['lin'][-1]['W'] + params['lin'][-1]['b']
    out, emb = _finalize(out_lin, final_emb)
    return (out.T, emb[None, :, :])


# speedup vs baseline: 40.0127x; 1.2424x over previous
import functools

import jax
import jax.numpy as jnp
from jax import lax
from jax.experimental import pallas as pl
from jax.experimental.pallas import tpu as pltpu
from jax.experimental.pallas import tpu_sc as plsc

N = 100000
NP = 100096
NT = NP // 16
E = 1600000
EROWS = E // 128
ERP = 12544
RPT = ERP // 16
CH = 2
NPAIR = RPT // (2 * CH)
DCH = 4
HID = 32
HALF = 16
NEG = -1.0e30

_f32 = jnp.float32
_i32 = jnp.int32

_MESH = plsc.VectorSubcoreMesh(core_axis_name="c", subcore_axis_name="s",
                               num_cores=2, num_subcores=16)

_GDN = lax.GatherDimensionNumbers(offset_dims=(), collapsed_slice_dims=(0,),
                                  start_index_map=(0,))


def _bcast_lane(v16, l):
    idx = jnp.full((16, 1), l, _i32)
    return lax.gather(v16, idx, _GDN, (1,),
                      mode=lax.GatherScatterMode.PROMISE_IN_BOUNDS)


def _zero_shared(acc2d, acc1d, z2, z1, nbase):
    def zz(i, carry):
        z2[i, :] = jnp.zeros((HALF,), _f32)
        return carry
    lax.fori_loop(0, 16, zz, 0)

    def z1f(i, carry):
        z1[pl.ds(i * 16, 16)] = jnp.zeros((16,), _f32)
        return carry
    lax.fori_loop(0, 25, z1f, 0)

    def zo(k, carry):
        pltpu.sync_copy(z2, acc2d.at[pl.ds(nbase + k * 16, 16)])
        return carry
    lax.fori_loop(0, NT // 16, zo, 0)

    def zs(k, carry):
        pltpu.sync_copy(z1, acc1d.at[pl.ds(nbase + k * 400, 400)])
        return carry
    lax.fori_loop(0, 15, zs, 0)
    pltpu.sync_copy(z1.at[pl.ds(0, 256)], acc1d.at[pl.ds(nbase + 6000, 256)])


def _deg_body(dst2, ones2, ea3, easum_o, cnt_o,
              easum_sh, cnt_sh, dst_v, ones_v, rows_v, z2, z1, sem_r):
    c = lax.axis_index("c")
    s = lax.axis_index("s")
    nbase = s * NT
    _zero_shared(easum_sh, cnt_sh, z2, z1, nbase)
    plsc.subcore_barrier()

    base_row = (c * 16 + s) * (ERP // 32)

    def chunk(ci, carry):
        row0 = base_row + ci * DCH
        pltpu.sync_copy(dst2.at[pl.ds(row0, DCH)], dst_v)
        pltpu.sync_copy(ones2.at[pl.ds(row0, DCH)], ones_v)
        pltpu.async_copy(ea3.at[pl.ds(row0 * 128, DCH * 128)], rows_v,
                         sem_r).wait()

        def rsc(j, carry2):
            pltpu.sync_copy(rows_v.at[pl.ds(j * 128, 128)],
                            easum_sh.at[dst_v.at[j]], add=True)
            pltpu.sync_copy(ones_v.at[j], cnt_sh.at[dst_v.at[j]], add=True)
            return carry2
        lax.fori_loop(0, DCH, rsc, 0)
        return carry
    lax.fori_loop(0, (ERP // 32) // DCH, chunk, 0)

    plsc.subcore_barrier()
    obase = c * NP + nbase
    pltpu.sync_copy(easum_sh.at[pl.ds(nbase, NT)], easum_o.at[pl.ds(obase, NT)])
    pltpu.sync_copy(cnt_sh.at[pl.ds(nbase, NT)], cnt_o.at[pl.ds(obase, NT)])


_deg_kernel = functools.partial(
    pl.kernel,
    compiler_params=pltpu.CompilerParams(use_tc_tiling_on_sc=False,
                                         needs_layout_passes=False),
    out_type=[jax.ShapeDtypeStruct((2 * NP, HALF), _f32),
              jax.ShapeDtypeStruct((2 * NP,), _f32)],
    mesh=_MESH,
    scratch_types=[
        pltpu.VMEM_SHARED((NP, HALF), _f32),
        pltpu.VMEM_SHARED((NP,), _f32),
        pltpu.VMEM((DCH, 128), _i32),
        pltpu.VMEM((DCH, 128), _f32),
        pltpu.VMEM((DCH * 128, HALF), _f32),
        pltpu.VMEM((16, HALF), _f32),
        pltpu.VMEM((400,), _f32),
        pltpu.SemaphoreType.DMA,
    ],
)(_deg_body)


def _edge_body(src2, dst2, ae2, asrc, adst, hflat,
               outh, s_out,
               out_sh, s_sh, asrc_sh, adst_sh,
               src_v, ae_v, as_v, ad_v, dst_v, si_v, a_v, rows_v, z2, z1,
               sem_g, sem_r0, sem_r1, sem_s0, sem_s1, sem_o0, sem_o1):
    c = lax.axis_index("c")
    s = lax.axis_index("s")
    nbase = s * NT
    _zero_shared(out_sh, s_sh, z2, z1, nbase)
    pltpu.sync_copy(asrc.at[pl.ds(nbase, NT)], asrc_sh.at[pl.ds(nbase, NT)])
    pltpu.sync_copy(adst.at[pl.ds(nbase, NT)], adst_sh.at[pl.ds(nbase, NT)])
    plsc.subcore_barrier()

    coff = c * N

    def load_fire(row0, b, sem):
        pltpu.sync_copy(src2.at[pl.ds(row0, CH)], src_v)
        pltpu.sync_copy(dst2.at[pl.ds(row0, CH)], dst_v.at[b])

        def off(jg, carry2):
            j = jg // 8
            g = (jg % 8) * 16
            si_v[b, j, pl.ds(g, 16)] = src_v[j, pl.ds(g, 16)] + coff
            return carry2
        lax.fori_loop(0, CH * 8, off, 0)
        dg = [pltpu.async_copy(asrc_sh.at[src_v.at[j]], as_v.at[j], sem_g)
              for j in range(CH)]
        dr = [pltpu.async_copy(hflat.at[si_v.at[b, j]], rows_v.at[b, j], sem)
              for j in range(CH)]
        return dg, dr

    def fire_adst(b):
        return [pltpu.async_copy(adst_sh.at[dst_v.at[b, j]], ad_v.at[j],
                                 sem_g) for j in range(CH)]

    def comp_phase(row0, b, da, sem):
        pltpu.sync_copy(ae2.at[pl.ds(row0, CH)], ae_v)
        for d in da:
            d.wait()

        def comp(jg, carry2):
            j = jg // 8
            g = (jg % 8) * 16
            v = (as_v[j, pl.ds(g, 16)] + ad_v[j, pl.ds(g, 16)]
                 + ae_v[j, pl.ds(g, 16)])
            v = jnp.where(v >= 0, v, 0.2 * v)
            a_v[b, j, pl.ds(g, 16)] = jnp.exp(v)
            return carry2
        lax.fori_loop(0, CH * 8, comp, 0)
        ds_ = []

        @pl.when(c == 0)
        def _():
            for j in range(CH):
                ds_.append(pltpu.async_copy(
                    a_v.at[b, j], s_sh.at[dst_v.at[b, j]], sem, add=True))
        return ds_

    def wgt_phase(b, dr, sem):
        for d in dr:
            d.wait()

        def wgt(jg, carry2):
            j = jg // 8
            g = (jg % 8) * 16
            w16 = a_v[b, j, pl.ds(g, 16)]
            for l in range(16):
                wb = _bcast_lane(w16, l)
                r = rows_v[b, j, g + l, :]
                rows_v[b, j, g + l, :] = r * wb
            return carry2
        lax.fori_loop(0, CH * 8, wgt, 0)
        return [pltpu.async_copy(rows_v.at[b, j], out_sh.at[dst_v.at[b, j]],
                                 sem, add=True) for j in range(CH)]

    def pair(ci, carry):
        rp = s * RPT + ci * (2 * CH)
        rq = rp + CH
        dgp, drp = load_fire(rp, 0, sem_r0)
        dap = fire_adst(0)
        dsp = comp_phase(rp, 0, dgp + dap, sem_s0)
        dgq, drq = load_fire(rq, 1, sem_r1)
        dop = wgt_phase(0, drp, sem_o0)
        daq = fire_adst(1)
        dsq = comp_phase(rq, 1, dgq + daq, sem_s1)
        doq = wgt_phase(1, drq, sem_o1)
        for d in dop + doq:
            d.wait()

        @pl.when(c == 0)
        def _():
            for d in dsp + dsq:
                d.wait()
        return carry
    lax.fori_loop(0, NPAIR, pair, 0)

    plsc.subcore_barrier()
    obase = c * NP + nbase
    pltpu.sync_copy(out_sh.at[pl.ds(nbase, NT)], outh.at[pl.ds(obase, NT)])

    @pl.when(c == 0)
    def _():
        pltpu.sync_copy(s_sh.at[pl.ds(nbase, NT)], s_out.at[pl.ds(nbase, NT)])


_edge_kernel = functools.partial(
    pl.kernel,
    compiler_params=pltpu.CompilerParams(use_tc_tiling_on_sc=False,
                                         needs_layout_passes=False),
    out_type=[jax.ShapeDtypeStruct((2 * NP, HALF), _f32),
              jax.ShapeDtypeStruct((NP,), _f32)],
    mesh=_MESH,
    scratch_types=[
        pltpu.VMEM_SHARED((NP, HALF), _f32),
        pltpu.VMEM_SHARED((NP,), _f32),
        pltpu.VMEM_SHARED((NP,), _f32),
        pltpu.VMEM_SHARED((NP,), _f32),
        pltpu.VMEM((CH, 128), _i32),
        pltpu.VMEM((CH, 128), _f32),
        pltpu.VMEM((CH, 128), _f32),
        pltpu.VMEM((CH, 128), _f32),
        pltpu.VMEM((2, CH, 128), _i32),
        pltpu.VMEM((2, CH, 128), _i32),
        pltpu.VMEM((2, CH, 128), _f32),
        pltpu.VMEM((2, CH, 128, HALF), _f32),
        pltpu.VMEM((16, HALF), _f32),
        pltpu.VMEM((400,), _f32),
        pltpu.SemaphoreType.DMA,
        pltpu.SemaphoreType.DMA,
        pltpu.SemaphoreType.DMA,
        pltpu.SemaphoreType.DMA,
        pltpu.SemaphoreType.DMA,
        pltpu.SemaphoreType.DMA,
        pltpu.SemaphoreType.DMA,
    ],
)(_edge_body)


def _sub_body(x_ref, x0_ref, e_ref, o_ref, eo_ref):
    o_ref[...] = x_ref[...] - x0_ref[...]
    eo_ref[...] = e_ref[...]


def _finalize(xx, emb):
    blk = 2000
    out, emb_o = pl.pallas_call(
        _sub_body,
        grid=(N // blk,),
        in_specs=[
            pl.BlockSpec((blk, 1), lambda i: (i, 0)),
            pl.BlockSpec((1, 1), lambda i: (0, 0)),
            pl.BlockSpec((blk, HID), lambda i: (i, 0)),
        ],
        out_specs=[
            pl.BlockSpec((blk, 1), lambda i: (i, 0)),
            pl.BlockSpec((blk, HID), lambda i: (i, 0)),
        ],
        out_shape=[
            jax.ShapeDtypeStruct((N, 1), jnp.float32),
            jax.ShapeDtypeStruct((N, HID), jnp.float32),
        ],
    )(xx, xx[0:1], emb)
    return out, emb_o


def _pad_rows(a2, val):
    pad = jnp.full((ERP - EROWS, 128), val, a2.dtype)
    return jnp.concatenate([a2, pad], axis=0)


def _padn(v):
    return jnp.concatenate([v, jnp.zeros((NP - N,), v.dtype)])


def _gat_sc(x, su2, du2, ae2, p, self_loop, loop_attr):
    h = x @ p['W']
    asrc = (h * p['att_src']).sum(-1)
    adst = (h * p['att_dst']).sum(-1)
    hflat = jnp.concatenate([h[:, :HALF], h[:, HALF:]], axis=0)
    outh, s_acc = _edge_kernel(su2, du2, ae2, _padn(asrc), _padn(adst), hflat)
    acc = jnp.concatenate([outh[:NP][:N], outh[NP:][:N]], axis=1)
    s_acc = s_acc[:N]
    if self_loop:
        he_l = loop_attr @ p['We']
        ael = (he_l * p['att_edge']).sum(-1)
        alpha_s = asrc + adst + ael
        alpha_s = jnp.where(alpha_s >= 0, alpha_s, 0.2 * alpha_s)
        es = jnp.exp(alpha_s)
        acc = acc + es[:, None] * h
        s_acc = s_acc + es
    out = acc / (s_acc + 1e-16)[:, None]
    return out + p['b']


def _aedge2(ea, p):
    he = ea @ p['We']
    ae = (he * p['att_edge']).sum(-1)
    return _pad_rows(ae.reshape(EROWS, 128), NEG)


def kernel(x, edge_index, edge_attr, params):
    src_u, dst_u = edge_index[0], edge_index[1]
    su2 = _pad_rows(src_u.reshape(EROWS, 128), 0)
    du2 = _pad_rows(dst_u.reshape(EROWS, 128), 0)
    ones2 = _pad_rows(jnp.ones((EROWS, 128), _f32), 0.0)
    ea3 = jnp.concatenate(
        [edge_attr, jnp.zeros((ERP * 128 - E, HALF), _f32)], axis=0)

    easum_p, cnt_p = _deg_kernel(du2, ones2, ea3)
    ea_sum = (easum_p[:NP] + easum_p[NP:])[:N]
    cnt = (cnt_p[:NP] + cnt_p[NP:])[:N]
    loop_attr = ea_sum / jnp.maximum(cnt, 1.0)[:, None]

    xu = x
    for i in range(3):
        p = params['toup'][i]
        xu = jax.nn.relu(_gat_sc(xu, su2, du2, _aedge2(edge_attr, p),
                                 p, True, loop_attr))
    p0 = params['todown'][0]
    ae_d0 = _aedge2(edge_attr, p0)
    xd = jax.nn.relu(_gat_sc(xu, du2, su2, ae_d0, p0, False, None))
    xx = xd + xu
    for i in range(2):
        xx = jax.nn.relu(xx @ params['lin'][i]['W'] + params['lin'][i]['b'])
        p = params['todown'][i]
        ae_d = ae_d0 if i == 0 else _aedge2(edge_attr, p)
        xd = jax.nn.relu(_gat_sc(xx, du2, su2, ae_d, p, False, None))
        xx = xd + xu
    final_emb = xx
    out_lin = xx @ params['lin'][-1]['W'] + params['lin'][-1]['b']
    out, emb = _finalize(out_lin, final_emb)
    return (out.T, emb[None, :, :])


# speedup vs baseline: 40.3968x; 1.0096x over previous
import functools

import jax
import jax.numpy as jnp
from jax import lax
from jax.experimental import pallas as pl
from jax.experimental.pallas import tpu as pltpu
from jax.experimental.pallas import tpu_sc as plsc

N = 100000
NP = 100096
NT = NP // 16
E = 1600000
EROWS = E // 128
ERP = 12544
RPT = ERP // 16
CH = 2
NPAIR = RPT // (2 * CH)
DCH = 4
HID = 32
HALF = 16
NEG = -1.0e30

_f32 = jnp.float32
_i32 = jnp.int32

_MESH = plsc.VectorSubcoreMesh(core_axis_name="c", subcore_axis_name="s",
                               num_cores=2, num_subcores=16)

_GDN = lax.GatherDimensionNumbers(offset_dims=(), collapsed_slice_dims=(0,),
                                  start_index_map=(0,))


def _bcast_lane(v16, l):
    idx = jnp.full((16, 1), l, _i32)
    return lax.gather(v16, idx, _GDN, (1,),
                      mode=lax.GatherScatterMode.PROMISE_IN_BOUNDS)


def _zero_shared(acc2d, acc1d, z2, z1, nbase):
    def zz(i, carry):
        z2[i, :] = jnp.zeros((HALF,), _f32)
        return carry
    lax.fori_loop(0, 16, zz, 0)

    def z1f(i, carry):
        z1[pl.ds(i * 16, 16)] = jnp.zeros((16,), _f32)
        return carry
    lax.fori_loop(0, 25, z1f, 0)

    def zo(k, carry):
        pltpu.sync_copy(z2, acc2d.at[pl.ds(nbase + k * 16, 16)])
        return carry
    lax.fori_loop(0, NT // 16, zo, 0)

    def zs(k, carry):
        pltpu.sync_copy(z1, acc1d.at[pl.ds(nbase + k * 400, 400)])
        return carry
    lax.fori_loop(0, 15, zs, 0)
    pltpu.sync_copy(z1.at[pl.ds(0, 256)], acc1d.at[pl.ds(nbase + 6000, 256)])


def _deg_body(dst2, ones2, ea3, easum_o, cnt_o,
              easum_sh, cnt_sh, dst_v, ones_v, rows_v, z2, z1,
              sem_r, sem_e0, sem_e1, sem_c0, sem_c1):
    c = lax.axis_index("c")
    s = lax.axis_index("s")
    nbase = s * NT
    _zero_shared(easum_sh, cnt_sh, z2, z1, nbase)
    plsc.subcore_barrier()

    base_row = (c * 16 + s) * (ERP // 32)

    def ddrain(b, sem_e, sem_c):
        for j in range(DCH):
            pltpu.make_async_copy(rows_v.at[b, pl.ds(j * 128, 128)],
                                  easum_sh.at[dst_v.at[b, j]], sem_e).wait()
            pltpu.make_async_copy(ones_v.at[b, j],
                                  cnt_sh.at[dst_v.at[b, j]], sem_c).wait()

    def half(row0, b, sem_e, sem_c):
        pltpu.sync_copy(dst2.at[pl.ds(row0, DCH)], dst_v.at[b])
        pltpu.sync_copy(ones2.at[pl.ds(row0, DCH)], ones_v.at[b])
        pltpu.async_copy(ea3.at[pl.ds(row0 * 128, DCH * 128)], rows_v.at[b],
                         sem_r).wait()
        for j in range(DCH):
            pltpu.async_copy(rows_v.at[b, pl.ds(j * 128, 128)],
                             easum_sh.at[dst_v.at[b, j]], sem_e, add=True)
            pltpu.async_copy(ones_v.at[b, j], cnt_sh.at[dst_v.at[b, j]],
                             sem_c, add=True)

    def chunk(ci, carry):
        @pl.when(ci > 0)
        def _():
            ddrain(0, sem_e0, sem_c0)
            ddrain(1, sem_e1, sem_c1)
        row0 = base_row + ci * 2 * DCH
        half(row0, 0, sem_e0, sem_c0)
        half(row0 + DCH, 1, sem_e1, sem_c1)
        return carry
    lax.fori_loop(0, (ERP // 32) // (2 * DCH), chunk, 0)
    ddrain(0, sem_e0, sem_c0)
    ddrain(1, sem_e1, sem_c1)

    plsc.subcore_barrier()
    obase = c * NP + nbase
    pltpu.sync_copy(easum_sh.at[pl.ds(nbase, NT)], easum_o.at[pl.ds(obase, NT)])
    pltpu.sync_copy(cnt_sh.at[pl.ds(nbase, NT)], cnt_o.at[pl.ds(obase, NT)])


_deg_kernel = functools.partial(
    pl.kernel,
    compiler_params=pltpu.CompilerParams(use_tc_tiling_on_sc=False,
                                         needs_layout_passes=False),
    out_type=[jax.ShapeDtypeStruct((2 * NP, HALF), _f32),
              jax.ShapeDtypeStruct((2 * NP,), _f32)],
    mesh=_MESH,
    scratch_types=[
        pltpu.VMEM_SHARED((NP, HALF), _f32),
        pltpu.VMEM_SHARED((NP,), _f32),
        pltpu.VMEM((2, DCH, 128), _i32),
        pltpu.VMEM((2, DCH, 128), _f32),
        pltpu.VMEM((2, DCH * 128, HALF), _f32),
        pltpu.VMEM((16, HALF), _f32),
        pltpu.VMEM((400,), _f32),
        pltpu.SemaphoreType.DMA,
        pltpu.SemaphoreType.DMA,
        pltpu.SemaphoreType.DMA,
        pltpu.SemaphoreType.DMA,
        pltpu.SemaphoreType.DMA,
    ],
)(_deg_body)


def _edge_body(src2, dst2, ae2, asrc, adst, hflat,
               outh, s_out,
               out_sh, s_sh, asrc_sh, adst_sh,
               src_v, ae_v, as_v, ad_v, dst_v, si_v, a_v, rows_v, z2, z1,
               sem_g, sem_r0, sem_r1, sem_s0, sem_s1, sem_o0, sem_o1):
    c = lax.axis_index("c")
    s = lax.axis_index("s")
    nbase = s * NT
    _zero_shared(out_sh, s_sh, z2, z1, nbase)
    pltpu.sync_copy(asrc.at[pl.ds(nbase, NT)], asrc_sh.at[pl.ds(nbase, NT)])
    pltpu.sync_copy(adst.at[pl.ds(nbase, NT)], adst_sh.at[pl.ds(nbase, NT)])
    plsc.subcore_barrier()

    coff = c * N

    def load_fire(row0, b, sem):
        pltpu.sync_copy(src2.at[pl.ds(row0, CH)], src_v)
        pltpu.sync_copy(dst2.at[pl.ds(row0, CH)], dst_v.at[b])

        def off(jg, carry2):
            j = jg // 8
            g = (jg % 8) * 16
            si_v[b, j, pl.ds(g, 16)] = src_v[j, pl.ds(g, 16)] + coff
            return carry2
        lax.fori_loop(0, CH * 8, off, 0)
        dg = [pltpu.async_copy(asrc_sh.at[src_v.at[j]], as_v.at[j], sem_g)
              for j in range(CH)]
        dr = [pltpu.async_copy(hflat.at[si_v.at[b, j]], rows_v.at[b, j], sem)
              for j in range(CH)]
        return dg, dr

    def fire_adst(b):
        return [pltpu.async_copy(adst_sh.at[dst_v.at[b, j]], ad_v.at[j],
                                 sem_g) for j in range(CH)]

    def comp_phase(row0, b, da, sem):
        pltpu.sync_copy(ae2.at[pl.ds(row0, CH)], ae_v)
        for d in da:
            d.wait()

        def comp(jg, carry2):
            j = jg // 8
            g = (jg % 8) * 16
            v = (as_v[j, pl.ds(g, 16)] + ad_v[j, pl.ds(g, 16)]
                 + ae_v[j, pl.ds(g, 16)])
            v = jnp.where(v >= 0, v, 0.2 * v)
            a_v[b, j, pl.ds(g, 16)] = jnp.exp(v)
            return carry2
        lax.fori_loop(0, CH * 8, comp, 0)
        ds_ = []

        @pl.when(c == 0)
        def _():
            for j in range(CH):
                ds_.append(pltpu.async_copy(
                    a_v.at[b, j], s_sh.at[dst_v.at[b, j]], sem, add=True))
        return ds_

    def wgt_phase(b, dr, sem):
        for d in dr:
            d.wait()

        def wgt(jg, carry2):
            j = jg // 8
            g = (jg % 8) * 16
            w16 = a_v[b, j, pl.ds(g, 16)]
            for l in range(16):
                wb = _bcast_lane(w16, l)
                r = rows_v[b, j, g + l, :]
                rows_v[b, j, g + l, :] = r * wb
            return carry2
        lax.fori_loop(0, CH * 8, wgt, 0)
        return [pltpu.async_copy(rows_v.at[b, j], out_sh.at[dst_v.at[b, j]],
                                 sem, add=True) for j in range(CH)]

    def drain(b, sem_sb, sem_ob):
        for j in range(CH):
            pltpu.make_async_copy(rows_v.at[b, j],
                                  out_sh.at[dst_v.at[b, j]], sem_ob).wait()

        @pl.when(c == 0)
        def _():
            for j in range(CH):
                pltpu.make_async_copy(a_v.at[b, j],
                                      s_sh.at[dst_v.at[b, j]], sem_sb).wait()

    def pair(ci, carry):
        @pl.when(ci > 0)
        def _():
            drain(0, sem_s0, sem_o0)
            drain(1, sem_s1, sem_o1)
        rp = s * RPT + ci * (2 * CH)
        rq = rp + CH
        dgp, drp = load_fire(rp, 0, sem_r0)
        dap = fire_adst(0)
        comp_phase(rp, 0, dgp + dap, sem_s0)
        dgq, drq = load_fire(rq, 1, sem_r1)
        wgt_phase(0, drp, sem_o0)
        daq = fire_adst(1)
        comp_phase(rq, 1, dgq + daq, sem_s1)
        wgt_phase(1, drq, sem_o1)
        return carry
    lax.fori_loop(0, NPAIR, pair, 0)
    drain(0, sem_s0, sem_o0)
    drain(1, sem_s1, sem_o1)

    plsc.subcore_barrier()
    obase = c * NP + nbase
    pltpu.sync_copy(out_sh.at[pl.ds(nbase, NT)], outh.at[pl.ds(obase, NT)])

    @pl.when(c == 0)
    def _():
        pltpu.sync_copy(s_sh.at[pl.ds(nbase, NT)], s_out.at[pl.ds(nbase, NT)])


_edge_kernel = functools.partial(
    pl.kernel,
    compiler_params=pltpu.CompilerParams(use_tc_tiling_on_sc=False,
                                         needs_layout_passes=False),
    out_type=[jax.ShapeDtypeStruct((2 * NP, HALF), _f32),
              jax.ShapeDtypeStruct((NP,), _f32)],
    mesh=_MESH,
    scratch_types=[
        pltpu.VMEM_SHARED((NP, HALF), _f32),
        pltpu.VMEM_SHARED((NP,), _f32),
        pltpu.VMEM_SHARED((NP,), _f32),
        pltpu.VMEM_SHARED((NP,), _f32),
        pltpu.VMEM((CH, 128), _i32),
        pltpu.VMEM((CH, 128), _f32),
        pltpu.VMEM((CH, 128), _f32),
        pltpu.VMEM((CH, 128), _f32),
        pltpu.VMEM((2, CH, 128), _i32),
        pltpu.VMEM((2, CH, 128), _i32),
        pltpu.VMEM((2, CH, 128), _f32),
        pltpu.VMEM((2, CH, 128, HALF), _f32),
        pltpu.VMEM((16, HALF), _f32),
        pltpu.VMEM((400,), _f32),
        pltpu.SemaphoreType.DMA,
        pltpu.SemaphoreType.DMA,
        pltpu.SemaphoreType.DMA,
        pltpu.SemaphoreType.DMA,
        pltpu.SemaphoreType.DMA,
        pltpu.SemaphoreType.DMA,
        pltpu.SemaphoreType.DMA,
    ],
)(_edge_body)


def _sub_body(x_ref, x0_ref, e_ref, o_ref, eo_ref):
    o_ref[...] = x_ref[...] - x0_ref[...]
    eo_ref[...] = e_ref[...]


def _finalize(xx, emb):
    blk = 2000
    out, emb_o = pl.pallas_call(
        _sub_body,
        grid=(N // blk,),
        in_specs=[
            pl.BlockSpec((blk, 1), lambda i: (i, 0)),
            pl.BlockSpec((1, 1), lambda i: (0, 0)),
            pl.BlockSpec((blk, HID), lambda i: (i, 0)),
        ],
        out_specs=[
            pl.BlockSpec((blk, 1), lambda i: (i, 0)),
            pl.BlockSpec((blk, HID), lambda i: (i, 0)),
        ],
        out_shape=[
            jax.ShapeDtypeStruct((N, 1), jnp.float32),
            jax.ShapeDtypeStruct((N, HID), jnp.float32),
        ],
    )(xx, xx[0:1], emb)
    return out, emb_o


def _pad_rows(a2, val):
    pad = jnp.full((ERP - EROWS, 128), val, a2.dtype)
    return jnp.concatenate([a2, pad], axis=0)


def _padn(v):
    return jnp.concatenate([v, jnp.zeros((NP - N,), v.dtype)])


def _gat_sc(x, su2, du2, ae2, p, self_loop, loop_attr):
    h = x @ p['W']
    asrc = (h * p['att_src']).sum(-1)
    adst = (h * p['att_dst']).sum(-1)
    hflat = jnp.concatenate([h[:, :HALF], h[:, HALF:]], axis=0)
    outh, s_acc = _edge_kernel(su2, du2, ae2, _padn(asrc), _padn(adst), hflat)
    acc = jnp.concatenate([outh[:NP][:N], outh[NP:][:N]], axis=1)
    s_acc = s_acc[:N]
    if self_loop:
        he_l = loop_attr @ p['We']
        ael = (he_l * p['att_edge']).sum(-1)
        alpha_s = asrc + adst + ael
        alpha_s = jnp.where(alpha_s >= 0, alpha_s, 0.2 * alpha_s)
        es = jnp.exp(alpha_s)
        acc = acc + es[:, None] * h
        s_acc = s_acc + es
    out = acc / (s_acc + 1e-16)[:, None]
    return out + p['b']


def _aedge2(ea, p):
    he = ea @ p['We']
    ae = (he * p['att_edge']).sum(-1)
    return _pad_rows(ae.reshape(EROWS, 128), NEG)


def kernel(x, edge_index, edge_attr, params):
    src_u, dst_u = edge_index[0], edge_index[1]
    su2 = _pad_rows(src_u.reshape(EROWS, 128), 0)
    du2 = _pad_rows(dst_u.reshape(EROWS, 128), 0)
    ones2 = _pad_rows(jnp.ones((EROWS, 128), _f32), 0.0)
    ea3 = jnp.concatenate(
        [edge_attr, jnp.zeros((ERP * 128 - E, HALF), _f32)], axis=0)

    easum_p, cnt_p = _deg_kernel(du2, ones2, ea3)
    ea_sum = (easum_p[:NP] + easum_p[NP:])[:N]
    cnt = (cnt_p[:NP] + cnt_p[NP:])[:N]
    loop_attr = ea_sum / jnp.maximum(cnt, 1.0)[:, None]

    xu = x
    for i in range(3):
        p = params['toup'][i]
        xu = jax.nn.relu(_gat_sc(xu, su2, du2, _aedge2(edge_attr, p),
                                 p, True, loop_attr))
    p0 = params['todown'][0]
    ae_d0 = _aedge2(edge_attr, p0)
    xd = jax.nn.relu(_gat_sc(xu, du2, su2, ae_d0, p0, False, None))
    xx = xd + xu
    for i in range(2):
        xx = jax.nn.relu(xx @ params['lin'][i]['W'] + params['lin'][i]['b'])
        p = params['todown'][i]
        ae_d = ae_d0 if i == 0 else _aedge2(edge_attr, p)
        xd = jax.nn.relu(_gat_sc(xx, du2, su2, ae_d, p, False, None))
        xx = xd + xu
    final_emb = xx
    out_lin = xx @ params['lin'][-1]['W'] + params['lin'][-1]['b']
    out, emb = _finalize(out_lin, final_emb)
    return (out.T, emb[None, :, :])


# speedup vs baseline: 40.9114x; 1.0127x over previous
import functools

import jax
import jax.numpy as jnp
from jax import lax
from jax.experimental import pallas as pl
from jax.experimental.pallas import tpu as pltpu
from jax.experimental.pallas import tpu_sc as plsc

N = 100000
NP = 100096
NT = NP // 16
E = 1600000
EROWS = E // 128
ERP = 12544
RPT = ERP // 16
CH = 2
NPAIR = RPT // (2 * CH)
DCH = 4
HID = 32
HALF = 16
NEG = -1.0e30

_f32 = jnp.float32
_i32 = jnp.int32

_MESH = plsc.VectorSubcoreMesh(core_axis_name="c", subcore_axis_name="s",
                               num_cores=2, num_subcores=16)

_GDN = lax.GatherDimensionNumbers(offset_dims=(), collapsed_slice_dims=(0,),
                                  start_index_map=(0,))


def _bcast_lane(v16, l):
    idx = jnp.full((16, 1), l, _i32)
    return lax.gather(v16, idx, _GDN, (1,),
                      mode=lax.GatherScatterMode.PROMISE_IN_BOUNDS)


def _zero_shared(acc2d, acc1d, z2, z1, nbase):
    def zz(i, carry):
        z2[i, :] = jnp.zeros((HALF,), _f32)
        return carry
    lax.fori_loop(0, 16, zz, 0)

    def z1f(i, carry):
        z1[pl.ds(i * 16, 16)] = jnp.zeros((16,), _f32)
        return carry
    lax.fori_loop(0, 25, z1f, 0)

    def zo(k, carry):
        pltpu.sync_copy(z2, acc2d.at[pl.ds(nbase + k * 16, 16)])
        return carry
    lax.fori_loop(0, NT // 16, zo, 0)

    def zs(k, carry):
        pltpu.sync_copy(z1, acc1d.at[pl.ds(nbase + k * 400, 400)])
        return carry
    lax.fori_loop(0, 15, zs, 0)
    pltpu.sync_copy(z1.at[pl.ds(0, 256)], acc1d.at[pl.ds(nbase + 6000, 256)])


def _deg_body(dst2, ones2, ea3, easum_o, cnt_o,
              easum_sh, cnt_sh, dst_v, ones_v, rows_v, z2, z1,
              sem_r, sem_e0, sem_e1, sem_c0, sem_c1):
    c = lax.axis_index("c")
    s = lax.axis_index("s")
    nbase = s * NT
    _zero_shared(easum_sh, cnt_sh, z2, z1, nbase)
    plsc.subcore_barrier()

    base_row = (c * 16 + s) * (ERP // 32)

    def ddrain(b, sem_e, sem_c):
        for j in range(DCH):
            pltpu.make_async_copy(rows_v.at[b, pl.ds(j * 128, 128)],
                                  easum_sh.at[dst_v.at[b, j]], sem_e).wait()
            pltpu.make_async_copy(ones_v.at[b, j],
                                  cnt_sh.at[dst_v.at[b, j]], sem_c).wait()

    def half(row0, b, sem_e, sem_c):
        pltpu.sync_copy(dst2.at[pl.ds(row0, DCH)], dst_v.at[b])
        pltpu.sync_copy(ones2.at[pl.ds(row0, DCH)], ones_v.at[b])
        pltpu.async_copy(ea3.at[pl.ds(row0 * 128, DCH * 128)], rows_v.at[b],
                         sem_r).wait()
        for j in range(DCH):
            pltpu.async_copy(rows_v.at[b, pl.ds(j * 128, 128)],
                             easum_sh.at[dst_v.at[b, j]], sem_e, add=True)
            pltpu.async_copy(ones_v.at[b, j], cnt_sh.at[dst_v.at[b, j]],
                             sem_c, add=True)

    def chunk(ci, carry):
        @pl.when(ci > 0)
        def _():
            ddrain(0, sem_e0, sem_c0)
            ddrain(1, sem_e1, sem_c1)
        row0 = base_row + ci * 2 * DCH
        half(row0, 0, sem_e0, sem_c0)
        half(row0 + DCH, 1, sem_e1, sem_c1)
        return carry
    lax.fori_loop(0, (ERP // 32) // (2 * DCH), chunk, 0)
    ddrain(0, sem_e0, sem_c0)
    ddrain(1, sem_e1, sem_c1)

    plsc.subcore_barrier()
    obase = c * NP + nbase
    pltpu.sync_copy(easum_sh.at[pl.ds(nbase, NT)], easum_o.at[pl.ds(obase, NT)])
    pltpu.sync_copy(cnt_sh.at[pl.ds(nbase, NT)], cnt_o.at[pl.ds(obase, NT)])


_deg_kernel = functools.partial(
    pl.kernel,
    compiler_params=pltpu.CompilerParams(use_tc_tiling_on_sc=False,
                                         needs_layout_passes=False),
    out_type=[jax.ShapeDtypeStruct((2 * NP, HALF), _f32),
              jax.ShapeDtypeStruct((2 * NP,), _f32)],
    mesh=_MESH,
    scratch_types=[
        pltpu.VMEM_SHARED((NP, HALF), _f32),
        pltpu.VMEM_SHARED((NP,), _f32),
        pltpu.VMEM((2, DCH, 128), _i32),
        pltpu.VMEM((2, DCH, 128), _f32),
        pltpu.VMEM((2, DCH * 128, HALF), _f32),
        pltpu.VMEM((16, HALF), _f32),
        pltpu.VMEM((400,), _f32),
        pltpu.SemaphoreType.DMA,
        pltpu.SemaphoreType.DMA,
        pltpu.SemaphoreType.DMA,
        pltpu.SemaphoreType.DMA,
        pltpu.SemaphoreType.DMA,
    ],
)(_deg_body)


def _edge_body(src2, dst2, ae2, asrc, adst, hflat,
               outh, s_out,
               out_sh, s_sh, asrc_sh, adst_sh,
               src_v, ae_v, as_v, ad_v, dst_v, si_v, a_v, rows_v, z2, z1,
               sem_g, sem_r0, sem_r1, sem_s0, sem_s1, sem_o0, sem_o1):
    c = lax.axis_index("c")
    s = lax.axis_index("s")
    nbase = s * NT
    _zero_shared(out_sh, s_sh, z2, z1, nbase)
    pltpu.sync_copy(asrc.at[pl.ds(nbase, NT)], asrc_sh.at[pl.ds(nbase, NT)])
    pltpu.sync_copy(adst.at[pl.ds(nbase, NT)], adst_sh.at[pl.ds(nbase, NT)])
    plsc.subcore_barrier()

    coff = c * N

    def load_fire(row0, b, sem):
        pltpu.sync_copy(src2.at[pl.ds(row0, CH)], src_v)
        pltpu.sync_copy(dst2.at[pl.ds(row0, CH)], dst_v.at[b])

        def off(jg, carry2):
            j = jg // 8
            g = (jg % 8) * 16
            si_v[b, j, pl.ds(g, 16)] = src_v[j, pl.ds(g, 16)] + coff
            return carry2
        lax.fori_loop(0, CH * 8, off, 0)
        dg = [pltpu.async_copy(asrc_sh.at[src_v.at[j]],
                               as_v.at[pl.ds(j * 128, 128)], sem_g)
              for j in range(CH)]
        dr = [pltpu.async_copy(hflat.at[si_v.at[b, j]],
                               rows_v.at[b, pl.ds(j * 128, 128)], sem)
              for j in range(CH)]
        return dg, dr

    def fire_adst(b):
        return [pltpu.async_copy(adst_sh.at[dst_v.at[b, j]],
                                 ad_v.at[pl.ds(j * 128, 128)], sem_g)
                for j in range(CH)]

    def comp_phase(row0, b, da, sem):
        pltpu.sync_copy(ae2.at[pl.ds(row0 * 128, CH * 128)], ae_v)
        for d in da:
            d.wait()

        def comp(jg, carry2):
            o = jg * 16
            v = (as_v[pl.ds(o, 16)] + ad_v[pl.ds(o, 16)]
                 + ae_v[pl.ds(o, 16)])
            v = jnp.where(v >= 0, v, 0.2 * v)
            a_v[b, pl.ds(o, 16)] = jnp.exp(v)
            return carry2
        lax.fori_loop(0, CH * 8, comp, 0)
        ds_ = []

        @pl.when(c == 0)
        def _():
            for j in range(CH):
                ds_.append(pltpu.async_copy(
                    a_v.at[b, pl.ds(j * 128, 128)],
                    s_sh.at[dst_v.at[b, j]], sem, add=True))
        return ds_

    def wgt_phase(b, dr, sem):
        for d in dr:
            d.wait()

        def wgt(jg, carry2):
            o = jg * 16
            w16 = a_v[b, pl.ds(o, 16)]
            for l in range(16):
                wb = _bcast_lane(w16, l)
                r = rows_v[b, o + l, :]
                rows_v[b, o + l, :] = r * wb
            return carry2
        lax.fori_loop(0, CH * 8, wgt, 0)
        return [pltpu.async_copy(rows_v.at[b, pl.ds(j * 128, 128)],
                                 out_sh.at[dst_v.at[b, j]], sem, add=True)
                for j in range(CH)]

    def drain(b, sem_sb, sem_ob):
        for j in range(CH):
            pltpu.make_async_copy(rows_v.at[b, pl.ds(j * 128, 128)],
                                  out_sh.at[dst_v.at[b, j]], sem_ob).wait()

        @pl.when(c == 0)
        def _():
            for j in range(CH):
                pltpu.make_async_copy(a_v.at[b, pl.ds(j * 128, 128)],
                                      s_sh.at[dst_v.at[b, j]], sem_sb).wait()

    def pair(ci, carry):
        @pl.when(ci > 0)
        def _():
            drain(0, sem_s0, sem_o0)
            drain(1, sem_s1, sem_o1)
        rp = s * RPT + ci * (2 * CH)
        rq = rp + CH
        dgp, drp = load_fire(rp, 0, sem_r0)
        dap = fire_adst(0)
        comp_phase(rp, 0, dgp + dap, sem_s0)
        dgq, drq = load_fire(rq, 1, sem_r1)
        wgt_phase(0, drp, sem_o0)
        daq = fire_adst(1)
        comp_phase(rq, 1, dgq + daq, sem_s1)
        wgt_phase(1, drq, sem_o1)
        return carry
    lax.fori_loop(0, NPAIR, pair, 0)
    drain(0, sem_s0, sem_o0)
    drain(1, sem_s1, sem_o1)

    plsc.subcore_barrier()
    obase = c * NP + nbase
    pltpu.sync_copy(out_sh.at[pl.ds(nbase, NT)], outh.at[pl.ds(obase, NT)])

    @pl.when(c == 0)
    def _():
        pltpu.sync_copy(s_sh.at[pl.ds(nbase, NT)], s_out.at[pl.ds(nbase, NT)])


_edge_kernel = functools.partial(
    pl.kernel,
    compiler_params=pltpu.CompilerParams(use_tc_tiling_on_sc=False,
                                         needs_layout_passes=False),
    out_type=[jax.ShapeDtypeStruct((2 * NP, HALF), _f32),
              jax.ShapeDtypeStruct((NP,), _f32)],
    mesh=_MESH,
    scratch_types=[
        pltpu.VMEM_SHARED((NP, HALF), _f32),
        pltpu.VMEM_SHARED((NP,), _f32),
        pltpu.VMEM_SHARED((NP,), _f32),
        pltpu.VMEM_SHARED((NP,), _f32),
        pltpu.VMEM((CH, 128), _i32),
        pltpu.VMEM((CH * 128,), _f32),
        pltpu.VMEM((CH * 128,), _f32),
        pltpu.VMEM((CH * 128,), _f32),
        pltpu.VMEM((2, CH, 128), _i32),
        pltpu.VMEM((2, CH, 128), _i32),
        pltpu.VMEM((2, CH * 128), _f32),
        pltpu.VMEM((2, CH * 128, HALF), _f32),
        pltpu.VMEM((16, HALF), _f32),
        pltpu.VMEM((400,), _f32),
        pltpu.SemaphoreType.DMA,
        pltpu.SemaphoreType.DMA,
        pltpu.SemaphoreType.DMA,
        pltpu.SemaphoreType.DMA,
        pltpu.SemaphoreType.DMA,
        pltpu.SemaphoreType.DMA,
        pltpu.SemaphoreType.DMA,
    ],
)(_edge_body)


def _sub_body(x_ref, x0_ref, e_ref, o_ref, eo_ref):
    o_ref[...] = x_ref[...] - x0_ref[...]
    eo_ref[...] = e_ref[...]


def _finalize(xx, emb):
    blk = 2000
    out, emb_o = pl.pallas_call(
        _sub_body,
        grid=(N // blk,),
        in_specs=[
            pl.BlockSpec((blk, 1), lambda i: (i, 0)),
            pl.BlockSpec((1, 1), lambda i: (0, 0)),
            pl.BlockSpec((blk, HID), lambda i: (i, 0)),
        ],
        out_specs=[
            pl.BlockSpec((blk, 1), lambda i: (i, 0)),
            pl.BlockSpec((blk, HID), lambda i: (i, 0)),
        ],
        out_shape=[
            jax.ShapeDtypeStruct((N, 1), jnp.float32),
            jax.ShapeDtypeStruct((N, HID), jnp.float32),
        ],
    )(xx, xx[0:1], emb)
    return out, emb_o


def _pad_rows(a2, val):
    pad = jnp.full((ERP - EROWS, 128), val, a2.dtype)
    return jnp.concatenate([a2, pad], axis=0)


def _padn(v):
    return jnp.concatenate([v, jnp.zeros((NP - N,), v.dtype)])


def _gat_sc(x, su2, du2, ae2, p, self_loop, loop_attr):
    h = x @ p['W']
    asrc = (h * p['att_src']).sum(-1)
    adst = (h * p['att_dst']).sum(-1)
    hflat = jnp.concatenate([h[:, :HALF], h[:, HALF:]], axis=0)
    outh, s_acc = _edge_kernel(su2, du2, ae2, _padn(asrc), _padn(adst), hflat)
    acc = jnp.concatenate([outh[:NP][:N], outh[NP:][:N]], axis=1)
    s_acc = s_acc[:N]
    if self_loop:
        he_l = loop_attr @ p['We']
        ael = (he_l * p['att_edge']).sum(-1)
        alpha_s = asrc + adst + ael
        alpha_s = jnp.where(alpha_s >= 0, alpha_s, 0.2 * alpha_s)
        es = jnp.exp(alpha_s)
        acc = acc + es[:, None] * h
        s_acc = s_acc + es
    out = acc / (s_acc + 1e-16)[:, None]
    return out + p['b']


def _aedge2(ea, p):
    he = ea @ p['We']
    ae = (he * p['att_edge']).sum(-1)
    return jnp.concatenate([ae, jnp.full((ERP * 128 - E,), NEG, _f32)])


def kernel(x, edge_index, edge_attr, params):
    src_u, dst_u = edge_index[0], edge_index[1]
    su2 = _pad_rows(src_u.reshape(EROWS, 128), 0)
    du2 = _pad_rows(dst_u.reshape(EROWS, 128), 0)
    ones2 = _pad_rows(jnp.ones((EROWS, 128), _f32), 0.0)
    ea3 = jnp.concatenate(
        [edge_attr, jnp.zeros((ERP * 128 - E, HALF), _f32)], axis=0)

    easum_p, cnt_p = _deg_kernel(du2, ones2, ea3)
    ea_sum = (easum_p[:NP] + easum_p[NP:])[:N]
    cnt = (cnt_p[:NP] + cnt_p[NP:])[:N]
    loop_attr = ea_sum / jnp.maximum(cnt, 1.0)[:, None]

    xu = x
    for i in range(3):
        p = params['toup'][i]
        xu = jax.nn.relu(_gat_sc(xu, su2, du2, _aedge2(edge_attr, p),
                                 p, True, loop_attr))
    p0 = params['todown'][0]
    ae_d0 = _aedge2(edge_attr, p0)
    xd = jax.nn.relu(_gat_sc(xu, du2, su2, ae_d0, p0, False, None))
    xx = xd + xu
    for i in range(2):
        xx = jax.nn.relu(xx @ params['lin'][i]['W'] + params['lin'][i]['b'])
        p = params['todown'][i]
        ae_d = ae_d0 if i == 0 else _aedge2(edge_attr, p)
        xd = jax.nn.relu(_gat_sc(xx, du2, su2, ae_d, p, False, None))
        xx = xd + xu
    final_emb = xx
    out_lin = xx @ params['lin'][-1]['W'] + params['lin'][-1]['b']
    out, emb = _finalize(out_lin, final_emb)
    return (out.T, emb[None, :, :])


# speedup vs baseline: 41.7129x; 1.0196x over previous
import functools

import jax
import jax.numpy as jnp
from jax import lax
from jax.experimental import pallas as pl
from jax.experimental.pallas import tpu as pltpu
from jax.experimental.pallas import tpu_sc as plsc

N = 100000
NP = 100096
NT = NP // 16
E = 1600000
EROWS = E // 128
ERP = 12544
RPT = ERP // 16
CH = 2
NPAIR = RPT // (2 * CH)
DCH = 4
HID = 32
HALF = 16
NEG = -1.0e30

_f32 = jnp.float32
_i32 = jnp.int32

_MESH = plsc.VectorSubcoreMesh(core_axis_name="c", subcore_axis_name="s",
                               num_cores=2, num_subcores=16)

_GDN = lax.GatherDimensionNumbers(offset_dims=(), collapsed_slice_dims=(0,),
                                  start_index_map=(0,))


def _bcast_lane(v16, l):
    idx = jnp.full((16, 1), l, _i32)
    return lax.gather(v16, idx, _GDN, (1,),
                      mode=lax.GatherScatterMode.PROMISE_IN_BOUNDS)


def _zero_shared(acc2d, acc1d, z2, z1, nbase):
    def zz(i, carry):
        z2[i, :] = jnp.zeros((HALF,), _f32)
        return carry
    lax.fori_loop(0, 16, zz, 0)

    def z1f(i, carry):
        z1[pl.ds(i * 16, 16)] = jnp.zeros((16,), _f32)
        return carry
    lax.fori_loop(0, 25, z1f, 0)

    def zo(k, carry):
        pltpu.sync_copy(z2, acc2d.at[pl.ds(nbase + k * 16, 16)])
        return carry
    lax.fori_loop(0, NT // 16, zo, 0)

    def zs(k, carry):
        pltpu.sync_copy(z1, acc1d.at[pl.ds(nbase + k * 400, 400)])
        return carry
    lax.fori_loop(0, 15, zs, 0)
    pltpu.sync_copy(z1.at[pl.ds(0, 256)], acc1d.at[pl.ds(nbase + 6000, 256)])


def _deg_body(dst2, ones2, ea3, easum_o, cnt_o,
              easum_sh, cnt_sh, dst_v, ones_v, rows_v, z2, z1,
              sem_r, sem_e0, sem_e1, sem_c0, sem_c1):
    c = lax.axis_index("c")
    s = lax.axis_index("s")
    nbase = s * NT
    _zero_shared(easum_sh, cnt_sh, z2, z1, nbase)
    plsc.subcore_barrier()

    base_row = (c * 16 + s) * (ERP // 32)

    def ddrain(b, sem_e, sem_c):
        for j in range(DCH):
            pltpu.make_async_copy(rows_v.at[b, pl.ds(j * 128, 128)],
                                  easum_sh.at[dst_v.at[b, j]], sem_e).wait()
            pltpu.make_async_copy(ones_v.at[b, j],
                                  cnt_sh.at[dst_v.at[b, j]], sem_c).wait()

    def half(row0, b, sem_e, sem_c):
        pltpu.sync_copy(dst2.at[pl.ds(row0, DCH)], dst_v.at[b])
        pltpu.sync_copy(ones2.at[pl.ds(row0, DCH)], ones_v.at[b])
        pltpu.async_copy(ea3.at[pl.ds(row0 * 128, DCH * 128)], rows_v.at[b],
                         sem_r).wait()
        for j in range(DCH):
            pltpu.async_copy(rows_v.at[b, pl.ds(j * 128, 128)],
                             easum_sh.at[dst_v.at[b, j]], sem_e, add=True)
            pltpu.async_copy(ones_v.at[b, j], cnt_sh.at[dst_v.at[b, j]],
                             sem_c, add=True)

    def chunk(ci, carry):
        @pl.when(ci > 0)
        def _():
            ddrain(0, sem_e0, sem_c0)
            ddrain(1, sem_e1, sem_c1)
        row0 = base_row + ci * 2 * DCH
        half(row0, 0, sem_e0, sem_c0)
        half(row0 + DCH, 1, sem_e1, sem_c1)
        return carry
    lax.fori_loop(0, (ERP // 32) // (2 * DCH), chunk, 0)
    ddrain(0, sem_e0, sem_c0)
    ddrain(1, sem_e1, sem_c1)

    plsc.subcore_barrier()
    obase = c * NP + nbase
    pltpu.sync_copy(easum_sh.at[pl.ds(nbase, NT)], easum_o.at[pl.ds(obase, NT)])
    pltpu.sync_copy(cnt_sh.at[pl.ds(nbase, NT)], cnt_o.at[pl.ds(obase, NT)])


_deg_kernel = functools.partial(
    pl.kernel,
    compiler_params=pltpu.CompilerParams(use_tc_tiling_on_sc=False,
                                         needs_layout_passes=False),
    out_type=[jax.ShapeDtypeStruct((2 * NP, HALF), _f32),
              jax.ShapeDtypeStruct((2 * NP,), _f32)],
    mesh=_MESH,
    scratch_types=[
        pltpu.VMEM_SHARED((NP, HALF), _f32),
        pltpu.VMEM_SHARED((NP,), _f32),
        pltpu.VMEM((2, DCH, 128), _i32),
        pltpu.VMEM((2, DCH, 128), _f32),
        pltpu.VMEM((2, DCH * 128, HALF), _f32),
        pltpu.VMEM((16, HALF), _f32),
        pltpu.VMEM((400,), _f32),
        pltpu.SemaphoreType.DMA,
        pltpu.SemaphoreType.DMA,
        pltpu.SemaphoreType.DMA,
        pltpu.SemaphoreType.DMA,
        pltpu.SemaphoreType.DMA,
    ],
)(_deg_body)


def _edge_body(src2, dst2, ae2, asrc, adst, hflat,
               outh, s_out,
               out_sh, s_sh, asrc_sh, adst_sh,
               src_v, ae_v, as_v, ad_v, dst_v, si_v, a_v, rows_v, z2, z1,
               sem_g, sem_r0, sem_r1, sem_s0, sem_s1, sem_o0, sem_o1):
    c = lax.axis_index("c")
    s = lax.axis_index("s")
    nbase = s * NT
    _zero_shared(out_sh, s_sh, z2, z1, nbase)
    pltpu.sync_copy(asrc.at[pl.ds(nbase, NT)], asrc_sh.at[pl.ds(nbase, NT)])
    pltpu.sync_copy(adst.at[pl.ds(nbase, NT)], adst_sh.at[pl.ds(nbase, NT)])
    plsc.subcore_barrier()

    coff = c * N

    def load_fire(row0, b, sem):
        pltpu.sync_copy(src2.at[pl.ds(row0, CH)], src_v)
        pltpu.sync_copy(dst2.at[pl.ds(row0, CH)], dst_v.at[b])

        @plsc.parallel_loop(0, CH * 8, step=1, unroll=4)
        def off(jg):
            j = jg // 8
            g = (jg % 8) * 16
            si_v[b, j, pl.ds(g, 16)] = src_v[j, pl.ds(g, 16)] + coff
        dg = [pltpu.async_copy(asrc_sh.at[src_v.at[j]],
                               as_v.at[pl.ds(j * 128, 128)], sem_g)
              for j in range(CH)]
        dr = [pltpu.async_copy(hflat.at[si_v.at[b, j]],
                               rows_v.at[b, pl.ds(j * 128, 128)], sem)
              for j in range(CH)]
        return dg, dr

    def fire_adst(b):
        return [pltpu.async_copy(adst_sh.at[dst_v.at[b, j]],
                                 ad_v.at[pl.ds(j * 128, 128)], sem_g)
                for j in range(CH)]

    def comp_phase(row0, b, da, sem):
        pltpu.sync_copy(ae2.at[pl.ds(row0 * 128, CH * 128)], ae_v)
        for d in da:
            d.wait()

        @plsc.parallel_loop(0, CH * 8, step=1, unroll=4)
        def comp(jg):
            o = jg * 16
            v = (as_v[pl.ds(o, 16)] + ad_v[pl.ds(o, 16)]
                 + ae_v[pl.ds(o, 16)])
            v = jnp.where(v >= 0, v, 0.2 * v)
            a_v[b, pl.ds(o, 16)] = jnp.exp(v)
        ds_ = []

        @pl.when(c == 0)
        def _():
            for j in range(CH):
                ds_.append(pltpu.async_copy(
                    a_v.at[b, pl.ds(j * 128, 128)],
                    s_sh.at[dst_v.at[b, j]], sem, add=True))
        return ds_

    def wgt_phase(b, dr, sem):
        for d in dr:
            d.wait()

        @plsc.parallel_loop(0, CH * 8, step=1, unroll=2)
        def wgt(jg):
            o = jg * 16
            w16 = a_v[b, pl.ds(o, 16)]
            for l in range(16):
                wb = _bcast_lane(w16, l)
                r = rows_v[b, o + l, :]
                rows_v[b, o + l, :] = r * wb
        return [pltpu.async_copy(rows_v.at[b, pl.ds(j * 128, 128)],
                                 out_sh.at[dst_v.at[b, j]], sem, add=True)
                for j in range(CH)]

    def drain(b, sem_sb, sem_ob):
        for j in range(CH):
            pltpu.make_async_copy(rows_v.at[b, pl.ds(j * 128, 128)],
                                  out_sh.at[dst_v.at[b, j]], sem_ob).wait()

        @pl.when(c == 0)
        def _():
            for j in range(CH):
                pltpu.make_async_copy(a_v.at[b, pl.ds(j * 128, 128)],
                                      s_sh.at[dst_v.at[b, j]], sem_sb).wait()

    def pair(ci, carry):
        @pl.when(ci > 0)
        def _():
            drain(0, sem_s0, sem_o0)
            drain(1, sem_s1, sem_o1)
        rp = s * RPT + ci * (2 * CH)
        rq = rp + CH
        dgp, drp = load_fire(rp, 0, sem_r0)
        dap = fire_adst(0)
        comp_phase(rp, 0, dgp + dap, sem_s0)
        dgq, drq = load_fire(rq, 1, sem_r1)
        wgt_phase(0, drp, sem_o0)
        daq = fire_adst(1)
        comp_phase(rq, 1, dgq + daq, sem_s1)
        wgt_phase(1, drq, sem_o1)
        return carry
    lax.fori_loop(0, NPAIR, pair, 0)
    drain(0, sem_s0, sem_o0)
    drain(1, sem_s1, sem_o1)

    plsc.subcore_barrier()
    obase = c * NP + nbase
    pltpu.sync_copy(out_sh.at[pl.ds(nbase, NT)], outh.at[pl.ds(obase, NT)])

    @pl.when(c == 0)
    def _():
        pltpu.sync_copy(s_sh.at[pl.ds(nbase, NT)], s_out.at[pl.ds(nbase, NT)])


_edge_kernel = functools.partial(
    pl.kernel,
    compiler_params=pltpu.CompilerParams(use_tc_tiling_on_sc=False,
                                         needs_layout_passes=False),
    out_type=[jax.ShapeDtypeStruct((2 * NP, HALF), _f32),
              jax.ShapeDtypeStruct((NP,), _f32)],
    mesh=_MESH,
    scratch_types=[
        pltpu.VMEM_SHARED((NP, HALF), _f32),
        pltpu.VMEM_SHARED((NP,), _f32),
        pltpu.VMEM_SHARED((NP,), _f32),
        pltpu.VMEM_SHARED((NP,), _f32),
        pltpu.VMEM((CH, 128), _i32),
        pltpu.VMEM((CH * 128,), _f32),
        pltpu.VMEM((CH * 128,), _f32),
        pltpu.VMEM((CH * 128,), _f32),
        pltpu.VMEM((2, CH, 128), _i32),
        pltpu.VMEM((2, CH, 128), _i32),
        pltpu.VMEM((2, CH * 128), _f32),
        pltpu.VMEM((2, CH * 128, HALF), _f32),
        pltpu.VMEM((16, HALF), _f32),
        pltpu.VMEM((400,), _f32),
        pltpu.SemaphoreType.DMA,
        pltpu.SemaphoreType.DMA,
        pltpu.SemaphoreType.DMA,
        pltpu.SemaphoreType.DMA,
        pltpu.SemaphoreType.DMA,
        pltpu.SemaphoreType.DMA,
        pltpu.SemaphoreType.DMA,
    ],
)(_edge_body)


def _sub_body(x_ref, x0_ref, e_ref, o_ref, eo_ref):
    o_ref[...] = x_ref[...] - x0_ref[...]
    eo_ref[...] = e_ref[...]


def _finalize(xx, emb):
    blk = 2000
    out, emb_o = pl.pallas_call(
        _sub_body,
        grid=(N // blk,),
        in_specs=[
            pl.BlockSpec((blk, 1), lambda i: (i, 0)),
            pl.BlockSpec((1, 1), lambda i: (0, 0)),
            pl.BlockSpec((blk, HID), lambda i: (i, 0)),
        ],
        out_specs=[
            pl.BlockSpec((blk, 1), lambda i: (i, 0)),
            pl.BlockSpec((blk, HID), lambda i: (i, 0)),
        ],
        out_shape=[
            jax.ShapeDtypeStruct((N, 1), jnp.float32),
            jax.ShapeDtypeStruct((N, HID), jnp.float32),
        ],
    )(xx, xx[0:1], emb)
    return out, emb_o


def _pad_rows(a2, val):
    pad = jnp.full((ERP - EROWS, 128), val, a2.dtype)
    return jnp.concatenate([a2, pad], axis=0)


def _padn(v):
    return jnp.concatenate([v, jnp.zeros((NP - N,), v.dtype)])


def _gat_sc(x, su2, du2, ae2, p, self_loop, loop_attr):
    h = x @ p['W']
    asrc = (h * p['att_src']).sum(-1)
    adst = (h * p['att_dst']).sum(-1)
    hflat = jnp.concatenate([h[:, :HALF], h[:, HALF:]], axis=0)
    outh, s_acc = _edge_kernel(su2, du2, ae2, _padn(asrc), _padn(adst), hflat)
    acc = jnp.concatenate([outh[:NP][:N], outh[NP:][:N]], axis=1)
    s_acc = s_acc[:N]
    if self_loop:
        he_l = loop_attr @ p['We']
        ael = (he_l * p['att_edge']).sum(-1)
        alpha_s = asrc + adst + ael
        alpha_s = jnp.where(alpha_s >= 0, alpha_s, 0.2 * alpha_s)
        es = jnp.exp(alpha_s)
        acc = acc + es[:, None] * h
        s_acc = s_acc + es
    out = acc / (s_acc + 1e-16)[:, None]
    return out + p['b']


def _aedge2(ea, p):
    he = ea @ p['We']
    ae = (he * p['att_edge']).sum(-1)
    return jnp.concatenate([ae, jnp.full((ERP * 128 - E,), NEG, _f32)])


def kernel(x, edge_index, edge_attr, params):
    src_u, dst_u = edge_index[0], edge_index[1]
    su2 = _pad_rows(src_u.reshape(EROWS, 128), 0)
    du2 = _pad_rows(dst_u.reshape(EROWS, 128), 0)
    ones2 = _pad_rows(jnp.ones((EROWS, 128), _f32), 0.0)
    ea3 = jnp.concatenate(
        [edge_attr, jnp.zeros((ERP * 128 - E, HALF), _f32)], axis=0)

    easum_p, cnt_p = _deg_kernel(du2, ones2, ea3)
    ea_sum = (easum_p[:NP] + easum_p[NP:])[:N]
    cnt = (cnt_p[:NP] + cnt_p[NP:])[:N]
    loop_attr = ea_sum / jnp.maximum(cnt, 1.0)[:, None]

    xu = x
    for i in range(3):
        p = params['toup'][i]
        xu = jax.nn.relu(_gat_sc(xu, su2, du2, _aedge2(edge_attr, p),
                                 p, True, loop_attr))
    p0 = params['todown'][0]
    ae_d0 = _aedge2(edge_attr, p0)
    xd = jax.nn.relu(_gat_sc(xu, du2, su2, ae_d0, p0, False, None))
    xx = xd + xu
    for i in range(2):
        xx = jax.nn.relu(xx @ params['lin'][i]['W'] + params['lin'][i]['b'])
        p = params['todown'][i]
        ae_d = ae_d0 if i == 0 else _aedge2(edge_attr, p)
        xd = jax.nn.relu(_gat_sc(xx, du2, su2, ae_d, p, False, None))
        xx = xd + xu
    final_emb = xx
    out_lin = xx @ params['lin'][-1]['W'] + params['lin'][-1]['b']
    out, emb = _finalize(out_lin, final_emb)
    return (out.T, emb[None, :, :])
